# Initial kernel scaffold; baseline (speedup 1.0000x reference)
import jax
import jax.numpy as jnp
from jax.experimental import pallas as pl


def kernel(x, edge_index, W_lin, b_lin):
    raise NotImplementedError("write your pallas kernel here")


# speedup vs baseline: 19.6679x; 19.6679x over previous
import functools

import jax
import jax.numpy as jnp
from jax import lax
from jax.experimental import pallas as pl
from jax.experimental.pallas import tpu as pltpu
from jax.experimental.pallas import tpu_sc as plsc

_NC = 2
_NS = 16
_NW = _NC * _NS
_BM = 1024


def _mesh():
    return plsc.VectorSubcoreMesh(core_axis_name="c", subcore_axis_name="s")


def _make_deg(npad, ep):
    nrow = npad // 128

    @functools.partial(
        pl.kernel,
        out_type=jax.ShapeDtypeStruct((2, _NW, nrow, 128), jnp.float32),
        mesh=_mesh(),
        compiler_params=pltpu.CompilerParams(needs_layout_passes=False),
        scratch_types=[
            pltpu.VMEM((ep,), jnp.int32),
            pltpu.VMEM((ep,), jnp.int32),
            pltpu.VMEM((nrow, 128), jnp.float32),
            pltpu.VMEM((nrow, 128), jnp.float32),
        ],
    )
    def deg_kernel(src_hbm, dst_hbm, out_hbm, src_v, dst_v, ho_v, hi_v):
        c = lax.axis_index("c")
        s = lax.axis_index("s")
        wid = s * _NC + c
        pltpu.sync_copy(src_hbm.at[wid], src_v)
        pltpu.sync_copy(dst_hbm.at[wid], dst_v)
        zeros = jnp.zeros((16,), jnp.float32)

        def zbody(r, carry):
            for j in range(8):
                ho_v[r, pl.ds(j * 16, 16)] = zeros
                hi_v[r, pl.ds(j * 16, 16)] = zeros
            return carry

        lax.fori_loop(0, nrow, zbody, 0)
        ones = jnp.ones((16,), jnp.float32)

        def body(i, carry):
            iv = src_v[pl.ds(i * 16, 16)]
            plsc.addupdate_scatter(
                ho_v, [lax.shift_right_logical(iv, 7),
                       lax.bitwise_and(iv, 127)], ones)
            jv = dst_v[pl.ds(i * 16, 16)]
            plsc.addupdate_scatter(
                hi_v, [lax.shift_right_logical(jv, 7),
                       lax.bitwise_and(jv, 127)], ones)
            return carry

        lax.fori_loop(0, ep // 16, body, 0)
        pltpu.sync_copy(ho_v, out_hbm.at[0, wid])
        pltpu.sync_copy(hi_v, out_hbm.at[1, wid])

    return deg_kernel


def _make_agg(npad, h, nch, ch):
    rp = npad // _NS

    @functools.partial(
        pl.kernel,
        out_type=jax.ShapeDtypeStruct((_NC, npad, h), jnp.float32),
        mesh=_mesh(),
        scratch_types=[
            pltpu.VMEM((nch, ch), jnp.int32),
            pltpu.VMEM((nch, ch), jnp.int32),
            pltpu.VMEM((ch, h), jnp.float32),
            pltpu.VMEM_SHARED((npad, h), jnp.float32),
            pltpu.SemaphoreType.DMA,
        ],
    )
    def agg_kernel(g_hbm, src_hbm, dst_hbm, zeros_hbm, out_hbm,
                   src_v, dst_v, rows_v, acc_sh, sem):
        c = lax.axis_index("c")
        s = lax.axis_index("s")
        wid = s * _NC + c
        pltpu.sync_copy(src_hbm.at[wid], src_v)
        pltpu.sync_copy(dst_hbm.at[wid], dst_v)
        pltpu.sync_copy(zeros_hbm, acc_sh.at[pl.ds(s * rp, rp)])
        plsc.subcore_barrier()

        def body(i, carry):
            pltpu.async_copy(g_hbm.at[src_v.at[i]], rows_v, sem).wait()
            pltpu.sync_copy(rows_v, acc_sh.at[dst_v.at[i]], add=True)
            return carry

        lax.fori_loop(0, nch, body, 0)
        plsc.subcore_barrier()
        sl = pl.ds(s * rp, rp)
        pltpu.sync_copy(acc_sh.at[sl], out_hbm.at[c, sl])

    return agg_kernel


def _cols_from_rows(mat):
    eye = (lax.broadcasted_iota(jnp.int32, (128, 128), 0) ==
           lax.broadcasted_iota(jnp.int32, (128, 128), 1)).astype(jnp.float32)
    chunks = [
        jnp.sum(jnp.broadcast_to(mat[r:r + 1, :], (128, 128)) * eye,
                axis=1, keepdims=True)
        for r in range(mat.shape[0])
    ]
    return jnp.concatenate(chunks, axis=0)


def _scale_col(deg_rows):
    col = _cols_from_rows(deg_rows)
    return jnp.where(col > 0, lax.rsqrt(col), 0.0)


def _lin_body(x_ref, w_ref, b_ref, degh_ref, o_ref):
    hmat = jnp.dot(x_ref[...], w_ref[...],
                   preferred_element_type=jnp.float32) + b_ref[...]
    d = jnp.sum(degh_ref[...], axis=1)
    o_ref[...] = hmat * _scale_col(d[0])


def _make_lin(n, d, h):
    return pl.pallas_call(
        _lin_body,
        grid=(pl.cdiv(n, _BM),),
        in_specs=[
            pl.BlockSpec((_BM, d), lambda m: (m, 0)),
            pl.BlockSpec((d, h), lambda m: (0, 0)),
            pl.BlockSpec((1, h), lambda m: (0, 0)),
            pl.BlockSpec((2, _NW, _BM // 128, 128), lambda m: (0, 0, m, 0)),
        ],
        out_specs=pl.BlockSpec((_BM, h), lambda m: (m, 0)),
        out_shape=jax.ShapeDtypeStruct((n, h), jnp.float32),
    )


def _comb_body(mid, p_ref, degh_ref, o_ref):
    d = jnp.sum(degh_ref[...], axis=1)
    scale = _scale_col(d[1])
    if mid:
        scale = scale * _scale_col(d[0])
    o_ref[...] = (p_ref[0] + p_ref[1]) * scale


def _make_comb(n, h, mid):
    return pl.pallas_call(
        functools.partial(_comb_body, mid),
        grid=(pl.cdiv(n, _BM),),
        in_specs=[
            pl.BlockSpec((_NC, _BM, h), lambda m: (0, m, 0)),
            pl.BlockSpec((2, _NW, _BM // 128, 128), lambda m: (0, 0, m, 0)),
        ],
        out_specs=pl.BlockSpec((_BM, h), lambda m: (m, 0)),
        out_shape=jax.ShapeDtypeStruct((n, h), jnp.float32),
    )


def kernel(x, edge_index, W_lin, b_lin):
    n, d = x.shape
    h = W_lin.shape[1]
    e = edge_index.shape[1]
    ep = e // _NW
    ch = 125
    nch = ep // ch
    npad = ((n + _NS * 128 - 1) // (_NS * 128)) * (_NS * 128)

    src = edge_index[0].reshape(_NW, nch, ch)
    dst = edge_index[1].reshape(_NW, nch, ch)
    srcf = edge_index[0].reshape(_NW, ep)
    dstf = edge_index[1].reshape(_NW, ep)
    zerosh = jnp.zeros((npad // _NS, h), jnp.float32)
    b2 = b_lin.reshape(1, h)

    degh = _make_deg(npad, ep)(srcf, dstf)
    agg = _make_agg(npad, h, nch, ch)
    g0 = _make_lin(n, d, h)(x, W_lin, b2, degh)
    p = agg(g0, src, dst, zerosh)
    g1 = _make_comb(n, h, True)(p, degh)
    q = agg(g1, src, dst, zerosh)
    return _make_comb(n, h, False)(q, degh)


# speedup vs baseline: 21.0856x; 1.0721x over previous
import functools

import jax
import jax.numpy as jnp
from jax import lax
from jax.experimental import pallas as pl
from jax.experimental.pallas import tpu as pltpu
from jax.experimental.pallas import tpu_sc as plsc

_NC = 2
_NS = 16
_NW = _NC * _NS
_BM = 1024


def _mesh():
    return plsc.VectorSubcoreMesh(core_axis_name="c", subcore_axis_name="s")


def _make_deg(npad, ep):
    nrow = npad // 128

    @functools.partial(
        pl.kernel,
        out_type=jax.ShapeDtypeStruct((2, _NW, nrow, 128), jnp.float32),
        mesh=_mesh(),
        compiler_params=pltpu.CompilerParams(needs_layout_passes=False),
        scratch_types=[
            pltpu.VMEM((ep,), jnp.int32),
            pltpu.VMEM((ep,), jnp.int32),
            pltpu.VMEM((nrow, 128), jnp.float32),
            pltpu.VMEM((nrow, 128), jnp.float32),
        ],
    )
    def deg_kernel(src_hbm, dst_hbm, out_hbm, src_v, dst_v, ho_v, hi_v):
        c = lax.axis_index("c")
        s = lax.axis_index("s")
        wid = s * _NC + c
        pltpu.sync_copy(src_hbm.at[wid], src_v)
        pltpu.sync_copy(dst_hbm.at[wid], dst_v)
        zeros = jnp.zeros((16,), jnp.float32)

        def zbody(r, carry):
            for j in range(8):
                ho_v[r, pl.ds(j * 16, 16)] = zeros
                hi_v[r, pl.ds(j * 16, 16)] = zeros
            return carry

        lax.fori_loop(0, nrow, zbody, 0)
        ones = jnp.ones((16,), jnp.float32)

        def body(i, carry):
            iv = src_v[pl.ds(i * 16, 16)]
            plsc.addupdate_scatter(
                ho_v, [lax.shift_right_logical(iv, 7),
                       lax.bitwise_and(iv, 127)], ones)
            jv = dst_v[pl.ds(i * 16, 16)]
            plsc.addupdate_scatter(
                hi_v, [lax.shift_right_logical(jv, 7),
                       lax.bitwise_and(jv, 127)], ones)
            return carry

        lax.fori_loop(0, ep // 16, body, 0)
        pltpu.sync_copy(ho_v, out_hbm.at[0, wid])
        pltpu.sync_copy(hi_v, out_hbm.at[1, wid])

    return deg_kernel


def _make_agg(npad, h, nch, ch):
    rp = npad // _NS

    @functools.partial(
        pl.kernel,
        out_type=jax.ShapeDtypeStruct((_NC, npad, h), jnp.float32),
        mesh=_mesh(),
        scratch_types=[
            pltpu.VMEM((nch * ch,), jnp.int32),
            pltpu.VMEM((nch, ch), jnp.int32),
            pltpu.VMEM((ch, h), jnp.float32),
            pltpu.VMEM((ch, h), jnp.float32),
            pltpu.VMEM_SHARED((npad, h), jnp.float32),
            pltpu.SemaphoreType.DMA,
            pltpu.SemaphoreType.DMA,
        ],
    )
    def agg_kernel(g_hbm, srcf_hbm, dst_hbm, zeros_hbm, out_hbm,
                   src_v, dst_v, rows0_v, rows1_v, acc_sh, sem0, sem1):
        c = lax.axis_index("c")
        s = lax.axis_index("s")
        wid = s * _NC + c
        pltpu.sync_copy(srcf_hbm.at[wid], src_v)
        pltpu.sync_copy(dst_hbm.at[wid], dst_v)
        pltpu.sync_copy(zeros_hbm, acc_sh.at[pl.ds(s * rp, rp)])
        plsc.subcore_barrier()

        def body(j, carry):
            i0 = j * 2
            d0 = pltpu.async_copy(
                g_hbm.at[src_v.at[pl.ds(i0 * ch, ch)]], rows0_v, sem0)
            d1 = pltpu.async_copy(
                g_hbm.at[src_v.at[pl.ds((i0 + 1) * ch, ch)]], rows1_v, sem1)
            d0.wait()
            pltpu.sync_copy(rows0_v, acc_sh.at[dst_v.at[i0]], add=True)
            d1.wait()
            pltpu.sync_copy(rows1_v, acc_sh.at[dst_v.at[i0 + 1]], add=True)
            return carry

        lax.fori_loop(0, nch // 2, body, 0)
        if nch % 2:
            i = nch - 1
            pltpu.async_copy(
                g_hbm.at[src_v.at[pl.ds(i * ch, ch)]], rows0_v, sem0).wait()
            pltpu.sync_copy(rows0_v, acc_sh.at[dst_v.at[i]], add=True)
        plsc.subcore_barrier()
        sl = pl.ds(s * rp, rp)
        pltpu.sync_copy(acc_sh.at[sl], out_hbm.at[c, sl])

    return agg_kernel


def _cols_from_rows(mat):
    eye = (lax.broadcasted_iota(jnp.int32, (128, 128), 0) ==
           lax.broadcasted_iota(jnp.int32, (128, 128), 1)).astype(jnp.float32)
    chunks = [
        jnp.sum(jnp.broadcast_to(mat[r:r + 1, :], (128, 128)) * eye,
                axis=1, keepdims=True)
        for r in range(mat.shape[0])
    ]
    return jnp.concatenate(chunks, axis=0)


def _scale_col(deg_rows):
    col = _cols_from_rows(deg_rows)
    return jnp.where(col > 0, lax.rsqrt(col), 0.0)


def _lin_body(x_ref, w_ref, b_ref, degh_ref, o_ref):
    hmat = jnp.dot(x_ref[...], w_ref[...],
                   preferred_element_type=jnp.float32) + b_ref[...]
    d = jnp.sum(degh_ref[...], axis=1)
    o_ref[...] = hmat * _scale_col(d[0])


def _make_lin(n, d, h):
    return pl.pallas_call(
        _lin_body,
        grid=(pl.cdiv(n, _BM),),
        in_specs=[
            pl.BlockSpec((_BM, d), lambda m: (m, 0)),
            pl.BlockSpec((d, h), lambda m: (0, 0)),
            pl.BlockSpec((1, h), lambda m: (0, 0)),
            pl.BlockSpec((2, _NW, _BM // 128, 128), lambda m: (0, 0, m, 0)),
        ],
        out_specs=pl.BlockSpec((_BM, h), lambda m: (m, 0)),
        out_shape=jax.ShapeDtypeStruct((n, h), jnp.float32),
    )


def _comb_body(mid, p_ref, degh_ref, o_ref):
    d = jnp.sum(degh_ref[...], axis=1)
    scale = _scale_col(d[1])
    if mid:
        scale = scale * _scale_col(d[0])
    o_ref[...] = (p_ref[0] + p_ref[1]) * scale


def _make_comb(n, h, mid):
    return pl.pallas_call(
        functools.partial(_comb_body, mid),
        grid=(pl.cdiv(n, _BM),),
        in_specs=[
            pl.BlockSpec((_NC, _BM, h), lambda m: (0, m, 0)),
            pl.BlockSpec((2, _NW, _BM // 128, 128), lambda m: (0, 0, m, 0)),
        ],
        out_specs=pl.BlockSpec((_BM, h), lambda m: (m, 0)),
        out_shape=jax.ShapeDtypeStruct((n, h), jnp.float32),
    )


def kernel(x, edge_index, W_lin, b_lin):
    n, d = x.shape
    h = W_lin.shape[1]
    e = edge_index.shape[1]
    ep = e // _NW
    ch = 80
    nch = ep // ch
    npad = ((n + _NS * 128 - 1) // (_NS * 128)) * (_NS * 128)

    dst = edge_index[1].reshape(_NW, nch, ch)
    srcf = edge_index[0].reshape(_NW, ep)
    dstf = edge_index[1].reshape(_NW, ep)
    zerosh = jnp.zeros((npad // _NS, h), jnp.float32)
    b2 = b_lin.reshape(1, h)

    degh = _make_deg(npad, ep)(srcf, dstf)
    agg = _make_agg(npad, h, nch, ch)
    g0 = _make_lin(n, d, h)(x, W_lin, b2, degh)
    p = agg(g0, srcf, dst, zerosh)
    g1 = _make_comb(n, h, True)(p, degh)
    q = agg(g1, srcf, dst, zerosh)
    return _make_comb(n, h, False)(q, degh)


# speedup vs baseline: 21.4946x; 1.0194x over previous
import functools

import jax
import jax.numpy as jnp
from jax import lax
from jax.experimental import pallas as pl
from jax.experimental.pallas import tpu as pltpu
from jax.experimental.pallas import tpu_sc as plsc

_NC = 2
_NS = 16
_NW = _NC * _NS
_BM = 1024


def _mesh():
    return plsc.VectorSubcoreMesh(core_axis_name="c", subcore_axis_name="s")


def _make_deg(npad, ep):
    nrow = npad // 128

    @functools.partial(
        pl.kernel,
        out_type=jax.ShapeDtypeStruct((2, _NW, nrow, 128), jnp.float32),
        mesh=_mesh(),
        compiler_params=pltpu.CompilerParams(needs_layout_passes=False),
        scratch_types=[
            pltpu.VMEM((ep,), jnp.int32),
            pltpu.VMEM((ep,), jnp.int32),
            pltpu.VMEM((nrow, 128), jnp.float32),
            pltpu.VMEM((nrow, 128), jnp.float32),
        ],
    )
    def deg_kernel(src_hbm, dst_hbm, out_hbm, src_v, dst_v, ho_v, hi_v):
        c = lax.axis_index("c")
        s = lax.axis_index("s")
        wid = s * _NC + c
        pltpu.sync_copy(src_hbm.at[wid], src_v)
        pltpu.sync_copy(dst_hbm.at[wid], dst_v)
        zeros = jnp.zeros((16,), jnp.float32)

        def zbody(r, carry):
            for j in range(8):
                ho_v[r, pl.ds(j * 16, 16)] = zeros
                hi_v[r, pl.ds(j * 16, 16)] = zeros
            return carry

        lax.fori_loop(0, nrow, zbody, 0)
        ones = jnp.ones((16,), jnp.float32)

        def body(i, carry):
            iv = src_v[pl.ds(i * 16, 16)]
            plsc.addupdate_scatter(
                ho_v, [lax.shift_right_logical(iv, 7),
                       lax.bitwise_and(iv, 127)], ones)
            jv = dst_v[pl.ds(i * 16, 16)]
            plsc.addupdate_scatter(
                hi_v, [lax.shift_right_logical(jv, 7),
                       lax.bitwise_and(jv, 127)], ones)
            return carry

        lax.fori_loop(0, ep // 16, body, 0)
        pltpu.sync_copy(ho_v, out_hbm.at[0, wid])
        pltpu.sync_copy(hi_v, out_hbm.at[1, wid])

    return deg_kernel


def _make_agg(npad, h, nch, ch):
    rp = npad // _NS

    @functools.partial(
        pl.kernel,
        out_type=jax.ShapeDtypeStruct((_NC, npad, h), jnp.float32),
        mesh=_mesh(),
        scratch_types=[
            pltpu.VMEM((nch * ch,), jnp.int32),
            pltpu.VMEM((nch, ch), jnp.int32),
            pltpu.VMEM((ch, h), jnp.float32),
            pltpu.VMEM((ch, h), jnp.float32),
            pltpu.VMEM_SHARED((npad, h), jnp.float32),
            pltpu.SemaphoreType.DMA,
            pltpu.SemaphoreType.DMA,
            pltpu.SemaphoreType.DMA,
        ],
    )
    def agg_kernel(g_hbm, srcf_hbm, dst_hbm, zeros_hbm, out_hbm,
                   src_v, dst_v, rows0_v, rows1_v, acc_sh, sem0, sem1, sems):
        c = lax.axis_index("c")
        s = lax.axis_index("s")
        wid = s * _NC + c
        pltpu.sync_copy(srcf_hbm.at[wid], src_v)
        pltpu.sync_copy(dst_hbm.at[wid], dst_v)
        pltpu.sync_copy(zeros_hbm, acc_sh.at[pl.ds(s * rp, rp)])
        plsc.subcore_barrier()

        def body(j, carry):
            i0 = j * 2
            d0 = pltpu.async_copy(
                g_hbm.at[src_v.at[pl.ds(i0 * ch, ch)]], rows0_v, sem0)
            d1 = pltpu.async_copy(
                g_hbm.at[src_v.at[pl.ds((i0 + 1) * ch, ch)]], rows1_v, sem1)
            d0.wait()
            s0 = pltpu.async_copy(rows0_v, acc_sh.at[dst_v.at[i0]], sems,
                                  add=True)
            d1.wait()
            pltpu.sync_copy(rows1_v, acc_sh.at[dst_v.at[i0 + 1]], add=True)
            s0.wait()
            return carry

        lax.fori_loop(0, nch // 2, body, 0)
        if nch % 2:
            i = nch - 1
            pltpu.async_copy(
                g_hbm.at[src_v.at[pl.ds(i * ch, ch)]], rows0_v, sem0).wait()
            pltpu.sync_copy(rows0_v, acc_sh.at[dst_v.at[i]], add=True)
        plsc.subcore_barrier()
        sl = pl.ds(s * rp, rp)
        pltpu.sync_copy(acc_sh.at[sl], out_hbm.at[c, sl])

    return agg_kernel


def _cols_from_rows(mat):
    eye = (lax.broadcasted_iota(jnp.int32, (128, 128), 0) ==
           lax.broadcasted_iota(jnp.int32, (128, 128), 1)).astype(jnp.float32)
    chunks = [
        jnp.sum(jnp.broadcast_to(mat[r:r + 1, :], (128, 128)) * eye,
                axis=1, keepdims=True)
        for r in range(mat.shape[0])
    ]
    return jnp.concatenate(chunks, axis=0)


def _scale_col(deg_rows):
    col = _cols_from_rows(deg_rows)
    return jnp.where(col > 0, lax.rsqrt(col), 0.0)


def _lin_body(x_ref, w_ref, b_ref, degh_ref, o_ref):
    hmat = jnp.dot(x_ref[...], w_ref[...],
                   preferred_element_type=jnp.float32) + b_ref[...]
    d = jnp.sum(degh_ref[...], axis=1)
    o_ref[...] = hmat * _scale_col(d[0])


def _make_lin(n, d, h):
    return pl.pallas_call(
        _lin_body,
        grid=(pl.cdiv(n, _BM),),
        in_specs=[
            pl.BlockSpec((_BM, d), lambda m: (m, 0)),
            pl.BlockSpec((d, h), lambda m: (0, 0)),
            pl.BlockSpec((1, h), lambda m: (0, 0)),
            pl.BlockSpec((2, _NW, _BM // 128, 128), lambda m: (0, 0, m, 0)),
        ],
        out_specs=pl.BlockSpec((_BM, h), lambda m: (m, 0)),
        out_shape=jax.ShapeDtypeStruct((n, h), jnp.float32),
    )


def _comb_body(mid, p_ref, degh_ref, o_ref):
    d = jnp.sum(degh_ref[...], axis=1)
    scale = _scale_col(d[1])
    if mid:
        scale = scale * _scale_col(d[0])
    o_ref[...] = (p_ref[0] + p_ref[1]) * scale


def _make_comb(n, h, mid):
    return pl.pallas_call(
        functools.partial(_comb_body, mid),
        grid=(pl.cdiv(n, _BM),),
        in_specs=[
            pl.BlockSpec((_NC, _BM, h), lambda m: (0, m, 0)),
            pl.BlockSpec((2, _NW, _BM // 128, 128), lambda m: (0, 0, m, 0)),
        ],
        out_specs=pl.BlockSpec((_BM, h), lambda m: (m, 0)),
        out_shape=jax.ShapeDtypeStruct((n, h), jnp.float32),
    )


def kernel(x, edge_index, W_lin, b_lin):
    n, d = x.shape
    h = W_lin.shape[1]
    e = edge_index.shape[1]
    ep = e // _NW
    ch = 80
    nch = ep // ch
    npad = ((n + _NS * 128 - 1) // (_NS * 128)) * (_NS * 128)

    dst = edge_index[1].reshape(_NW, nch, ch)
    srcf = edge_index[0].reshape(_NW, ep)
    dstf = edge_index[1].reshape(_NW, ep)
    zerosh = jnp.zeros((npad // _NS, h), jnp.float32)
    b2 = b_lin.reshape(1, h)

    degh = _make_deg(npad, ep)(srcf, dstf)
    agg = _make_agg(npad, h, nch, ch)
    g0 = _make_lin(n, d, h)(x, W_lin, b2, degh)
    p = agg(g0, srcf, dst, zerosh)
    g1 = _make_comb(n, h, True)(p, degh)
    q = agg(g1, srcf, dst, zerosh)
    return _make_comb(n, h, False)(q, degh)
